# baseline (speedup 1.0000x reference)
import jax
import jax.numpy as jnp
from jax import lax
from jax.experimental import pallas as pl
from jax.experimental.pallas import tpu as pltpu
from jax.experimental.pallas import tpu_sc as plsc

N_USERS = 10000
NP = 10240
D = 128
E = 320000
NC = 2
NS = 16
NW = NC * NS
C = 64
NBG = 4
NBS = 3
UNROLL = 12
IR = 8
DP = D // 2
N0CH = 228
N1CH = 96
TCH = NS * (N0CH + N1CH)
EPAD = TCH * C
EPW1 = EPAD // NW
RPS = NP // NS
LANES = 16

_SC_PARAMS = pltpu.CompilerParams(needs_layout_passes=False)
_SC_PARAMS_UNTILED = pltpu.CompilerParams(
    needs_layout_passes=False, use_tc_tiling_on_sc=False)

import numpy as _np
_perm = _np.zeros(D, _np.int32)
for _c in range(D // 32):
    for _i in range(16):
        _perm[32 * _c + 2 * _i] = 32 * _c + _i
        _perm[32 * _c + 2 * _i + 1] = 32 * _c + 16 + _i
_PERM = _perm


def _deg_body(dst_hbm, w_hbm, deg_out):
    cid = lax.axis_index("c")
    sid = lax.axis_index("s")
    wid = sid * NC + cid
    zero16 = jnp.zeros((LANES,), jnp.float32)

    def inner(dst_all, w_all, deg_l):
        def zb(i, _):
            deg_l[pl.ds(i * LANES, LANES)] = zero16
            return 0

        lax.fori_loop(0, NP // LANES, zb, 0)

        pltpu.sync_copy(dst_hbm.at[wid], dst_all)
        pltpu.sync_copy(w_hbm.at[wid], w_all)

        def grp(i, _):
            dv = dst_all[pl.ds(i * LANES, LANES)]
            wv = w_all[pl.ds(i * LANES, LANES)]
            plsc.addupdate_scatter(deg_l, [dv], wv)
            return 0

        lax.fori_loop(0, EPW1 // LANES, grp, 0)
        pltpu.sync_copy(deg_l, deg_out.at[wid])

    pl.run_scoped(
        inner,
        pltpu.VMEM((EPW1,), jnp.int32),
        pltpu.VMEM((EPW1,), jnp.float32),
        pltpu.VMEM((NP,), jnp.float32),
    )

_deg_kernel = pl.kernel(
    _deg_body,
    out_type=jax.ShapeDtypeStruct((NW, NP), jnp.float32),
    mesh=plsc.VectorSubcoreMesh(core_axis_name="c", subcore_axis_name="s"),
    compiler_params=_SC_PARAMS,
)


def _agg_body(h2p_hbm, src_hbm, dst_hbm, w_hbm, out_hbm, acc,
              gs0, gs1, gs2, gs3, ss0, ss1, ss2, isem):
    cid = lax.axis_index("c")
    sid = lax.axis_index("s")
    gsems = (gs0, gs1, gs2, gs3)
    ssems = (ss0, ss1, ss2)
    zero16 = jnp.zeros((LANES,), jnp.float32)
    zidx = jnp.zeros((LANES,), jnp.int32)
    rbase = sid * RPS
    nch = jnp.where(cid == 0, N0CH, N1CH)
    cbase = jnp.where(cid == 0, sid * N0CH, NS * N0CH + sid * N1CH)

    def inner(rows_bf, rows_f, srcr, dstr, wr):
        def idx_prefetch(j):
            s = lax.rem(j, IR)
            pltpu.async_copy(src_hbm.at[cbase + j], srcr.at[s], isem)
            pltpu.async_copy(dst_hbm.at[cbase + j], dstr.at[s], isem)
            pltpu.async_copy(w_hbm.at[cbase + j], wr.at[s], isem)

        def idx_drain(j):
            s = lax.rem(j, IR)
            pltpu.make_async_copy(src_hbm.at[cbase + j], srcr.at[s], isem).wait()
            pltpu.make_async_copy(dst_hbm.at[cbase + j], dstr.at[s], isem).wait()
            pltpu.make_async_copy(w_hbm.at[cbase + j], wr.at[s], isem).wait()

        def start_gather(j, b):
            s = lax.rem(j, IR)
            pltpu.async_copy(h2p_hbm.at[srcr.at[s]], rows_bf.at[b], gsems[b])

        def wait_gather(j, b):
            s = lax.rem(j, IR)
            pltpu.make_async_copy(h2p_hbm.at[srcr.at[s]], rows_bf.at[b],
                                  gsems[b]).wait()

        def start_scatter(j, b):
            s = lax.rem(j, IR)
            pltpu.async_copy(rows_f.at[b], acc.at[dstr.at[s]], ssems[b], add=True)

        def wait_scatter(j, b):
            s = lax.rem(j, IR)
            pltpu.make_async_copy(rows_f.at[b], acc.at[dstr.at[s]],
                                  ssems[b]).wait()

        def scale_convert(j, bin_, bout):
            s = lax.rem(j, IR)

            def row(r, _):
                wspl = plsc.load_gather(wr, [zidx + s, zidx + r])
                for cc in range(DP // LANES):
                    wv = rows_bf[bin_, r, pl.ds(cc * LANES, LANES)]
                    bfv = plsc.bitcast(wv, jnp.bfloat16)
                    av, bv = plsc.unpack(bfv, format=plsc.PackFormat.INTERLEAVED)
                    rows_f[bout, r, pl.ds(cc * 2 * LANES, LANES)] = av * wspl
                    rows_f[bout, r, pl.ds((cc * 2 + 1) * LANES, LANES)] = bv * wspl
                return 0

            lax.fori_loop(0, C, row, 0)

        def zr(r, _):
            for cc in range(D // LANES):
                rows_f[0, r, pl.ds(cc * LANES, LANES)] = zero16
            return 0

        lax.fori_loop(0, C, zr, 0)
        for k in range(RPS // C):
            pltpu.sync_copy(rows_f.at[0], acc.at[pl.ds(rbase + k * C, C)])
        plsc.subcore_barrier()

        for j in range(5):
            idx_prefetch(j)
        idx_drain(0)
        start_gather(0, 0)
        idx_drain(1)
        start_gather(1, 1)

        def outer(jj, _):
            j0 = jj * UNROLL
            for u in range(UNROLL):
                j = j0 + u
                b4 = u % NBG
                b3 = u % NBS
                wait_gather(j, b4)

                @pl.when(j >= NBS)
                def _():
                    wait_scatter(j - NBS, b3)

                scale_convert(j, b4, b3)
                start_scatter(j, b3)
                jg = j + 2

                @pl.when(jg < nch)
                def _():
                    idx_drain(jg)
                    start_gather(jg, (u + 2) % NBG)

                jp = j + 5

                @pl.when(jp < nch)
                def _():
                    idx_prefetch(jp)
            return 0

        lax.fori_loop(0, nch // UNROLL, outer, 0)
        for v in range(NBS):
            wait_scatter(nch - NBS + v, v)
        plsc.subcore_barrier()

        def cp(k, _):
            pltpu.sync_copy(acc.at[pl.ds(rbase + k * C, C)], rows_f.at[0])
            pltpu.sync_copy(rows_f.at[0], out_hbm.at[cid, pl.ds(rbase + k * C, C)])
            return 0

        lax.fori_loop(0, RPS // C, cp, 0)

    pl.run_scoped(
        inner,
        pltpu.VMEM((NBG, C, DP), jnp.int32),
        pltpu.VMEM((NBS, C, D), jnp.float32),
        pltpu.VMEM((IR, C), jnp.int32),
        pltpu.VMEM((IR, C), jnp.int32),
        pltpu.VMEM((IR, C), jnp.float32),
    )


_agg_kernel = pl.kernel(
    _agg_body,
    out_type=jax.ShapeDtypeStruct((NC, NP, D), jnp.float32),
    mesh=plsc.VectorSubcoreMesh(core_axis_name="c", subcore_axis_name="s"),
    scratch_types=[
        pltpu.VMEM_SHARED((NP, D), jnp.float32),
        pltpu.SemaphoreType.DMA,
        pltpu.SemaphoreType.DMA,
        pltpu.SemaphoreType.DMA,
        pltpu.SemaphoreType.DMA,
        pltpu.SemaphoreType.DMA,
        pltpu.SemaphoreType.DMA,
        pltpu.SemaphoreType.DMA,
        pltpu.SemaphoreType.DMA,
    ],
    compiler_params=_SC_PARAMS_UNTILED,
)

BR = 2048
_GRID = NP // BR


def _h2_body(deg32_ref, x_ref, w_ref, h2_ref):
    deg = jnp.sum(deg32_ref[...], axis=0) + 1.0
    dis = lax.rsqrt(deg)
    h = jnp.dot(x_ref[...], w_ref[...], preferred_element_type=jnp.float32)
    h2_ref[...] = h * dis[:, None]


_h2_call = pl.pallas_call(
    _h2_body,
    grid=(_GRID,),
    in_specs=[
        pl.BlockSpec((NW, BR), lambda i: (0, i)),
        pl.BlockSpec((BR, D), lambda i: (i, 0)),
        pl.BlockSpec((D, D), lambda i: (0, 0)),
    ],
    out_specs=pl.BlockSpec((BR, D), lambda i: (i, 0)),
    out_shape=jax.ShapeDtypeStruct((NP, D), jnp.float32),
)


def _out_body(p_ref, h2_ref, deg32_ref, b_ref, o_ref):
    deg = jnp.sum(deg32_ref[...], axis=0) + 1.0
    dis = lax.rsqrt(deg)
    agg = p_ref[0] + p_ref[1] + h2_ref[...]
    o_ref[...] = jnp.maximum(agg * dis[:, None] + b_ref[...], 0.0)


_out_call = pl.pallas_call(
    _out_body,
    grid=(_GRID,),
    in_specs=[
        pl.BlockSpec((NC, BR, D), lambda i: (0, i, 0)),
        pl.BlockSpec((BR, D), lambda i: (i, 0)),
        pl.BlockSpec((NW, BR), lambda i: (0, i)),
        pl.BlockSpec((1, D), lambda i: (0, 0)),
    ],
    out_specs=pl.BlockSpec((BR, D), lambda i: (i, 0)),
    out_shape=jax.ShapeDtypeStruct((NP, D), jnp.float32),
)


def kernel(x, edge_index, edge_weight, W, b, item_embeddings):
    pad = EPAD - E
    src_p = jnp.pad(edge_index[0], (0, pad)).reshape(TCH, C)
    dst_p = jnp.pad(edge_index[1], (0, pad)).reshape(TCH, C)
    w_p = jnp.pad(edge_weight, (0, pad)).reshape(TCH, C)
    x_p = jnp.pad(x, ((0, NP - N_USERS), (0, 0)))
    dst_flat = dst_p.reshape(NW, EPW1)
    w_flat = w_p.reshape(NW, EPW1)
    deg32 = _deg_kernel(dst_flat, w_flat)
    h2 = _h2_call(deg32, x_p, W)
    h2p = lax.bitcast_convert_type(
        h2[:, _PERM].astype(jnp.bfloat16).reshape(NP, DP, 2), jnp.int32)
    p = _agg_kernel(h2p, src_p, dst_p, w_p)
    user_h = _out_call(p, h2, deg32, b.reshape(1, D))
    return user_h[:N_USERS], item_embeddings


# speedup vs baseline: 1.4365x; 1.4365x over previous
import jax
import jax.numpy as jnp
from jax import lax
from jax.experimental import pallas as pl
from jax.experimental.pallas import tpu as pltpu
from jax.experimental.pallas import tpu_sc as plsc

N_USERS = 10000
NP = 10240
D = 128
E = 320000
NC = 2
NS = 16
NW = NC * NS
C = 64
NB = 4
IR = 8
N0CH = 232
N1CH = 84
TCH = NS * (N0CH + N1CH)
EPAD = TCH * C
EPW1 = EPAD // NW
RPS = NP // NS
LANES = 16

_SC_PARAMS = pltpu.CompilerParams(needs_layout_passes=False)


def _deg_body(dst_hbm, w_hbm, deg_out):
    cid = lax.axis_index("c")
    sid = lax.axis_index("s")
    wid = sid * NC + cid
    zero16 = jnp.zeros((LANES,), jnp.float32)

    def inner(dst_all, w_all, deg_l):
        def zb(i, _):
            deg_l[pl.ds(i * LANES, LANES)] = zero16
            return 0

        lax.fori_loop(0, NP // LANES, zb, 0)

        pltpu.sync_copy(dst_hbm.at[wid], dst_all)
        pltpu.sync_copy(w_hbm.at[wid], w_all)

        def grp(i, _):
            dv = dst_all[pl.ds(i * LANES, LANES)]
            wv = w_all[pl.ds(i * LANES, LANES)]
            plsc.addupdate_scatter(deg_l, [dv], wv)
            return 0

        lax.fori_loop(0, EPW1 // LANES, grp, 0)
        pltpu.sync_copy(deg_l, deg_out.at[wid])

    pl.run_scoped(
        inner,
        pltpu.VMEM((EPW1,), jnp.int32),
        pltpu.VMEM((EPW1,), jnp.float32),
        pltpu.VMEM((NP,), jnp.float32),
    )

_deg_kernel = pl.kernel(
    _deg_body,
    out_type=jax.ShapeDtypeStruct((NW, NP), jnp.float32),
    mesh=plsc.VectorSubcoreMesh(core_axis_name="c", subcore_axis_name="s"),
    compiler_params=_SC_PARAMS,
)


def _agg_body(h2_hbm, src_hbm, dst_hbm, w_hbm, out_hbm, acc,
              gs0, gs1, gs2, gs3, ss0, ss1, ss2, ss3, isem):
    cid = lax.axis_index("c")
    sid = lax.axis_index("s")
    gsems = (gs0, gs1, gs2, gs3)
    ssems = (ss0, ss1, ss2, ss3)
    zero16 = jnp.zeros((LANES,), jnp.float32)
    zidx = jnp.zeros((LANES,), jnp.int32)
    rbase = sid * RPS
    nch = jnp.where(cid == 0, N0CH, N1CH)
    cbase = jnp.where(cid == 0, sid * N0CH, NS * N0CH + sid * N1CH)

    def inner(rows, srcr, dstr, wr):
        def idx_prefetch(j):
            s = lax.rem(j, IR)
            pltpu.async_copy(src_hbm.at[cbase + j], srcr.at[s], isem)
            pltpu.async_copy(dst_hbm.at[cbase + j], dstr.at[s], isem)
            pltpu.async_copy(w_hbm.at[cbase + j], wr.at[s], isem)

        def idx_drain(j):
            s = lax.rem(j, IR)
            pltpu.make_async_copy(src_hbm.at[cbase + j], srcr.at[s], isem).wait()
            pltpu.make_async_copy(dst_hbm.at[cbase + j], dstr.at[s], isem).wait()
            pltpu.make_async_copy(w_hbm.at[cbase + j], wr.at[s], isem).wait()

        def start_gather(j, b):
            s = lax.rem(j, IR)
            pltpu.async_copy(h2_hbm.at[srcr.at[s]], rows.at[b], gsems[b])

        def wait_gather(j, b):
            s = lax.rem(j, IR)
            pltpu.make_async_copy(h2_hbm.at[srcr.at[s]], rows.at[b], gsems[b]).wait()

        def start_scatter(j, b):
            s = lax.rem(j, IR)
            pltpu.async_copy(rows.at[b], acc.at[dstr.at[s]], ssems[b], add=True)

        def wait_scatter(j, b):
            s = lax.rem(j, IR)
            pltpu.make_async_copy(rows.at[b], acc.at[dstr.at[s]], ssems[b]).wait()

        def scale(j, b):
            s = lax.rem(j, IR)

            @plsc.parallel_loop(0, C, 1, unroll=8)
            def _row(r):
                wspl = plsc.load_gather(wr, [zidx + s, zidx + r])
                for cc in range(D // LANES):
                    sl = pl.ds(cc * LANES, LANES)
                    rows[b, r, sl] = rows[b, r, sl] * wspl

        def zr(r, _):
            for cc in range(D // LANES):
                rows[0, r, pl.ds(cc * LANES, LANES)] = zero16
            return 0

        lax.fori_loop(0, C, zr, 0)
        for k in range(RPS // C):
            pltpu.sync_copy(rows.at[0], acc.at[pl.ds(rbase + k * C, C)])
        plsc.subcore_barrier()

        for j in range(6):
            idx_prefetch(j)
        idx_drain(0)
        start_gather(0, 0)
        idx_drain(1)
        start_gather(1, 1)

        def outer(jj, _):
            j0 = jj * NB
            for u in range(NB):
                j = j0 + u
                b = u
                wait_gather(j, b)
                scale(j, b)
                start_scatter(j, b)
                jg = j + 2
                bg = (u + 2) % NB

                @pl.when(jg < nch)
                def _():
                    @pl.when(jg >= NB)
                    def _():
                        wait_scatter(jg - NB, bg)

                    idx_drain(jg)
                    start_gather(jg, bg)

                jp = j + 6

                @pl.when(jp < nch)
                def _():
                    idx_prefetch(jp)
            return 0

        lax.fori_loop(0, nch // NB, outer, 0)
        for u in range(NB):
            wait_scatter(nch - NB + u, u)
        plsc.subcore_barrier()

        def cp(k, _):
            pltpu.sync_copy(acc.at[pl.ds(rbase + k * C, C)], rows.at[0])
            pltpu.sync_copy(rows.at[0], out_hbm.at[cid, pl.ds(rbase + k * C, C)])
            return 0

        lax.fori_loop(0, RPS // C, cp, 0)

    pl.run_scoped(
        inner,
        pltpu.VMEM((NB, C, D), jnp.float32),
        pltpu.VMEM((IR, C), jnp.int32),
        pltpu.VMEM((IR, C), jnp.int32),
        pltpu.VMEM((IR, C), jnp.float32),
    )


_agg_kernel = pl.kernel(
    _agg_body,
    out_type=jax.ShapeDtypeStruct((NC, NP, D), jnp.float32),
    mesh=plsc.VectorSubcoreMesh(core_axis_name="c", subcore_axis_name="s"),
    scratch_types=[
        pltpu.VMEM_SHARED((NP, D), jnp.float32),
        pltpu.SemaphoreType.DMA,
        pltpu.SemaphoreType.DMA,
        pltpu.SemaphoreType.DMA,
        pltpu.SemaphoreType.DMA,
        pltpu.SemaphoreType.DMA,
        pltpu.SemaphoreType.DMA,
        pltpu.SemaphoreType.DMA,
        pltpu.SemaphoreType.DMA,
        pltpu.SemaphoreType.DMA,
    ],
    compiler_params=_SC_PARAMS,
)

BR = 2048
_GRID = NP // BR


def _h2_body(deg32_ref, x_ref, w_ref, h2_ref):
    deg = jnp.sum(deg32_ref[...], axis=0) + 1.0
    dis = lax.rsqrt(deg)
    h = jnp.dot(x_ref[...], w_ref[...], preferred_element_type=jnp.float32)
    h2_ref[...] = h * dis[:, None]


_h2_call = pl.pallas_call(
    _h2_body,
    grid=(_GRID,),
    in_specs=[
        pl.BlockSpec((NW, BR), lambda i: (0, i)),
        pl.BlockSpec((BR, D), lambda i: (i, 0)),
        pl.BlockSpec((D, D), lambda i: (0, 0)),
    ],
    out_specs=pl.BlockSpec((BR, D), lambda i: (i, 0)),
    out_shape=jax.ShapeDtypeStruct((NP, D), jnp.float32),
)


def _out_body(p_ref, h2_ref, deg32_ref, b_ref, o_ref):
    deg = jnp.sum(deg32_ref[...], axis=0) + 1.0
    dis = lax.rsqrt(deg)
    agg = p_ref[0] + p_ref[1] + h2_ref[...]
    o_ref[...] = jnp.maximum(agg * dis[:, None] + b_ref[...], 0.0)


_out_call = pl.pallas_call(
    _out_body,
    grid=(_GRID,),
    in_specs=[
        pl.BlockSpec((NC, BR, D), lambda i: (0, i, 0)),
        pl.BlockSpec((BR, D), lambda i: (i, 0)),
        pl.BlockSpec((NW, BR), lambda i: (0, i)),
        pl.BlockSpec((1, D), lambda i: (0, 0)),
    ],
    out_specs=pl.BlockSpec((BR, D), lambda i: (i, 0)),
    out_shape=jax.ShapeDtypeStruct((NP, D), jnp.float32),
)


def kernel(x, edge_index, edge_weight, W, b, item_embeddings):
    pad = EPAD - E
    src_p = jnp.pad(edge_index[0], (0, pad)).reshape(TCH, C)
    dst_p = jnp.pad(edge_index[1], (0, pad)).reshape(TCH, C)
    w_p = jnp.pad(edge_weight, (0, pad)).reshape(TCH, C)
    x_p = jnp.pad(x, ((0, NP - N_USERS), (0, 0)))
    dst_flat = dst_p.reshape(NW, EPW1)
    w_flat = w_p.reshape(NW, EPW1)
    deg32 = _deg_kernel(dst_flat, w_flat)
    h2 = _h2_call(deg32, x_p, W)
    p = _agg_kernel(h2, src_p, dst_p, w_p)
    user_h = _out_call(p, h2, deg32, b.reshape(1, D))
    return user_h[:N_USERS], item_embeddings


# speedup vs baseline: 1.4554x; 1.0131x over previous
import jax
import jax.numpy as jnp
from jax import lax
from jax.experimental import pallas as pl
from jax.experimental.pallas import tpu as pltpu
from jax.experimental.pallas import tpu_sc as plsc

N_USERS = 10000
NP = 10240
D = 128
E = 320000
NC = 2
NS = 16
NW = NC * NS
C = 64
NB = 4
IR = 8
N0CH = 236
N1CH = 80
TCH = NS * (N0CH + N1CH)
EPAD = TCH * C
EPW1 = EPAD // NW
RPS = NP // NS
LANES = 16

_SC_PARAMS = pltpu.CompilerParams(needs_layout_passes=False)


def _deg_body(dst_hbm, w_hbm, deg_out):
    cid = lax.axis_index("c")
    sid = lax.axis_index("s")
    wid = sid * NC + cid
    zero16 = jnp.zeros((LANES,), jnp.float32)

    def inner(dst_all, w_all, deg_l):
        def zb(i, _):
            deg_l[pl.ds(i * LANES, LANES)] = zero16
            return 0

        lax.fori_loop(0, NP // LANES, zb, 0)

        pltpu.sync_copy(dst_hbm.at[wid], dst_all)
        pltpu.sync_copy(w_hbm.at[wid], w_all)

        def grp(i, _):
            dv = dst_all[pl.ds(i * LANES, LANES)]
            wv = w_all[pl.ds(i * LANES, LANES)]
            plsc.addupdate_scatter(deg_l, [dv], wv)
            return 0

        lax.fori_loop(0, EPW1 // LANES, grp, 0)
        pltpu.sync_copy(deg_l, deg_out.at[wid])

    pl.run_scoped(
        inner,
        pltpu.VMEM((EPW1,), jnp.int32),
        pltpu.VMEM((EPW1,), jnp.float32),
        pltpu.VMEM((NP,), jnp.float32),
    )

_deg_kernel = pl.kernel(
    _deg_body,
    out_type=jax.ShapeDtypeStruct((NW, NP), jnp.float32),
    mesh=plsc.VectorSubcoreMesh(core_axis_name="c", subcore_axis_name="s"),
    compiler_params=_SC_PARAMS,
)


def _agg_body(h2_hbm, src_hbm, dst_hbm, w_hbm, out_hbm, acc,
              gs0, gs1, gs2, gs3, ss0, ss1, ss2, ss3, isem):
    cid = lax.axis_index("c")
    sid = lax.axis_index("s")
    gsems = (gs0, gs1, gs2, gs3)
    ssems = (ss0, ss1, ss2, ss3)
    zero16 = jnp.zeros((LANES,), jnp.float32)
    zidx = jnp.zeros((LANES,), jnp.int32)
    rbase = sid * RPS
    nch = jnp.where(cid == 0, N0CH, N1CH)
    cbase = jnp.where(cid == 0, sid * N0CH, NS * N0CH + sid * N1CH)

    def inner(rows, srcr, dstr, wr):
        def idx_prefetch(j):
            s = lax.rem(j, IR)
            pltpu.async_copy(src_hbm.at[cbase + j], srcr.at[s], isem)
            pltpu.async_copy(dst_hbm.at[cbase + j], dstr.at[s], isem)
            pltpu.async_copy(w_hbm.at[cbase + j], wr.at[s], isem)

        def idx_drain(j):
            s = lax.rem(j, IR)
            pltpu.make_async_copy(src_hbm.at[cbase + j], srcr.at[s], isem).wait()
            pltpu.make_async_copy(dst_hbm.at[cbase + j], dstr.at[s], isem).wait()
            pltpu.make_async_copy(w_hbm.at[cbase + j], wr.at[s], isem).wait()

        def start_gather(j, b):
            s = lax.rem(j, IR)
            pltpu.async_copy(h2_hbm.at[srcr.at[s]], rows.at[b], gsems[b])

        def wait_gather(j, b):
            s = lax.rem(j, IR)
            pltpu.make_async_copy(h2_hbm.at[srcr.at[s]], rows.at[b], gsems[b]).wait()

        def start_scatter(j, b):
            s = lax.rem(j, IR)
            pltpu.async_copy(rows.at[b], acc.at[dstr.at[s]], ssems[b], add=True)

        def wait_scatter(j, b):
            s = lax.rem(j, IR)
            pltpu.make_async_copy(rows.at[b], acc.at[dstr.at[s]], ssems[b]).wait()

        def scale(j, b):
            s = lax.rem(j, IR)

            def row(r, _):
                wspl = plsc.load_gather(wr, [zidx + s, zidx + r])
                for cc in range(D // LANES):
                    sl = pl.ds(cc * LANES, LANES)
                    rows[b, r, sl] = rows[b, r, sl] * wspl
                return 0

            lax.fori_loop(0, C, row, 0)

        def zr(r, _):
            for cc in range(D // LANES):
                rows[0, r, pl.ds(cc * LANES, LANES)] = zero16
            return 0

        lax.fori_loop(0, C, zr, 0)
        for k in range(RPS // C):
            pltpu.sync_copy(rows.at[0], acc.at[pl.ds(rbase + k * C, C)])
        plsc.subcore_barrier()

        for j in range(6):
            idx_prefetch(j)
        idx_drain(0)
        start_gather(0, 0)
        idx_drain(1)
        start_gather(1, 1)

        def outer(jj, _):
            j0 = jj * NB
            for u in range(NB):
                j = j0 + u
                b = u
                wait_gather(j, b)
                scale(j, b)
                start_scatter(j, b)
                jg = j + 2
                bg = (u + 2) % NB

                @pl.when(jg < nch)
                def _():
                    @pl.when(jg >= NB)
                    def _():
                        wait_scatter(jg - NB, bg)

                    idx_drain(jg)
                    start_gather(jg, bg)

                jp = j + 6

                @pl.when(jp < nch)
                def _():
                    idx_prefetch(jp)
            return 0

        lax.fori_loop(0, nch // NB, outer, 0)
        for u in range(NB):
            wait_scatter(nch - NB + u, u)
        plsc.subcore_barrier()

        def cp(k, _):
            pltpu.sync_copy(acc.at[pl.ds(rbase + k * C, C)], rows.at[0])
            pltpu.sync_copy(rows.at[0], out_hbm.at[cid, pl.ds(rbase + k * C, C)])
            return 0

        lax.fori_loop(0, RPS // C, cp, 0)

    pl.run_scoped(
        inner,
        pltpu.VMEM((NB, C, D), jnp.float32),
        pltpu.VMEM((IR, C), jnp.int32),
        pltpu.VMEM((IR, C), jnp.int32),
        pltpu.VMEM((IR, C), jnp.float32),
    )


_agg_kernel = pl.kernel(
    _agg_body,
    out_type=jax.ShapeDtypeStruct((NC, NP, D), jnp.float32),
    mesh=plsc.VectorSubcoreMesh(core_axis_name="c", subcore_axis_name="s"),
    scratch_types=[
        pltpu.VMEM_SHARED((NP, D), jnp.float32),
        pltpu.SemaphoreType.DMA,
        pltpu.SemaphoreType.DMA,
        pltpu.SemaphoreType.DMA,
        pltpu.SemaphoreType.DMA,
        pltpu.SemaphoreType.DMA,
        pltpu.SemaphoreType.DMA,
        pltpu.SemaphoreType.DMA,
        pltpu.SemaphoreType.DMA,
        pltpu.SemaphoreType.DMA,
    ],
    compiler_params=_SC_PARAMS,
)

BR = 2048
_GRID = NP // BR


def _h2_body(deg32_ref, x_ref, w_ref, h2_ref):
    deg = jnp.sum(deg32_ref[...], axis=0) + 1.0
    dis = lax.rsqrt(deg)
    h = jnp.dot(x_ref[...], w_ref[...], preferred_element_type=jnp.float32)
    h2_ref[...] = h * dis[:, None]


_h2_call = pl.pallas_call(
    _h2_body,
    grid=(_GRID,),
    in_specs=[
        pl.BlockSpec((NW, BR), lambda i: (0, i)),
        pl.BlockSpec((BR, D), lambda i: (i, 0)),
        pl.BlockSpec((D, D), lambda i: (0, 0)),
    ],
    out_specs=pl.BlockSpec((BR, D), lambda i: (i, 0)),
    out_shape=jax.ShapeDtypeStruct((NP, D), jnp.float32),
)


def _out_body(p_ref, h2_ref, deg32_ref, b_ref, o_ref):
    deg = jnp.sum(deg32_ref[...], axis=0) + 1.0
    dis = lax.rsqrt(deg)
    agg = p_ref[0] + p_ref[1] + h2_ref[...]
    o_ref[...] = jnp.maximum(agg * dis[:, None] + b_ref[...], 0.0)


_out_call = pl.pallas_call(
    _out_body,
    grid=(_GRID,),
    in_specs=[
        pl.BlockSpec((NC, BR, D), lambda i: (0, i, 0)),
        pl.BlockSpec((BR, D), lambda i: (i, 0)),
        pl.BlockSpec((NW, BR), lambda i: (0, i)),
        pl.BlockSpec((1, D), lambda i: (0, 0)),
    ],
    out_specs=pl.BlockSpec((BR, D), lambda i: (i, 0)),
    out_shape=jax.ShapeDtypeStruct((NP, D), jnp.float32),
)


def kernel(x, edge_index, edge_weight, W, b, item_embeddings):
    pad = EPAD - E
    src_p = jnp.pad(edge_index[0], (0, pad)).reshape(TCH, C)
    dst_p = jnp.pad(edge_index[1], (0, pad)).reshape(TCH, C)
    w_p = jnp.pad(edge_weight, (0, pad)).reshape(TCH, C)
    x_p = jnp.pad(x, ((0, NP - N_USERS), (0, 0)))
    dst_flat = dst_p.reshape(NW, EPW1)
    w_flat = w_p.reshape(NW, EPW1)
    deg32 = _deg_kernel(dst_flat, w_flat)
    h2 = _h2_call(deg32, x_p, W)
    p = _agg_kernel(h2, src_p, dst_p, w_p)
    user_h = _out_call(p, h2, deg32, b.reshape(1, D))
    return user_h[:N_USERS], item_embeddings


# speedup vs baseline: 1.4667x; 1.0077x over previous
import jax
import jax.numpy as jnp
from jax import lax
from jax.experimental import pallas as pl
from jax.experimental.pallas import tpu as pltpu
from jax.experimental.pallas import tpu_sc as plsc

N_USERS = 10000
NP = 10240
D = 128
E = 320000
NC = 2
NS = 16
NW = NC * NS
C = 64
NB = 4
IR = 8
N0CH = 240
N1CH = 76
TCH = NS * (N0CH + N1CH)
EPAD = TCH * C
EPW1 = EPAD // NW
RPS = NP // NS
LANES = 16

_SC_PARAMS = pltpu.CompilerParams(needs_layout_passes=False)


def _deg_body(dst_hbm, w_hbm, deg_out):
    cid = lax.axis_index("c")
    sid = lax.axis_index("s")
    wid = sid * NC + cid
    zero16 = jnp.zeros((LANES,), jnp.float32)

    def inner(dst_all, w_all, deg_l):
        def zb(i, _):
            deg_l[pl.ds(i * LANES, LANES)] = zero16
            return 0

        lax.fori_loop(0, NP // LANES, zb, 0)

        pltpu.sync_copy(dst_hbm.at[wid], dst_all)
        pltpu.sync_copy(w_hbm.at[wid], w_all)

        def grp(i, _):
            dv = dst_all[pl.ds(i * LANES, LANES)]
            wv = w_all[pl.ds(i * LANES, LANES)]
            plsc.addupdate_scatter(deg_l, [dv], wv)
            return 0

        lax.fori_loop(0, EPW1 // LANES, grp, 0)
        pltpu.sync_copy(deg_l, deg_out.at[wid])

    pl.run_scoped(
        inner,
        pltpu.VMEM((EPW1,), jnp.int32),
        pltpu.VMEM((EPW1,), jnp.float32),
        pltpu.VMEM((NP,), jnp.float32),
    )

_deg_kernel = pl.kernel(
    _deg_body,
    out_type=jax.ShapeDtypeStruct((NW, NP), jnp.float32),
    mesh=plsc.VectorSubcoreMesh(core_axis_name="c", subcore_axis_name="s"),
    compiler_params=_SC_PARAMS,
)


def _agg_body(h2_hbm, src_hbm, dst_hbm, w_hbm, out_hbm, acc,
              gs0, gs1, gs2, gs3, ss0, ss1, ss2, ss3, isem):
    cid = lax.axis_index("c")
    sid = lax.axis_index("s")
    gsems = (gs0, gs1, gs2, gs3)
    ssems = (ss0, ss1, ss2, ss3)
    zero16 = jnp.zeros((LANES,), jnp.float32)
    zidx = jnp.zeros((LANES,), jnp.int32)
    rbase = sid * RPS
    nch = jnp.where(cid == 0, N0CH, N1CH)
    cbase = jnp.where(cid == 0, sid * N0CH, NS * N0CH + sid * N1CH)

    def inner(rows, srcr, dstr, wr):
        def idx_prefetch(j):
            s = lax.rem(j, IR)
            pltpu.async_copy(src_hbm.at[cbase + j], srcr.at[s], isem)
            pltpu.async_copy(dst_hbm.at[cbase + j], dstr.at[s], isem)
            pltpu.async_copy(w_hbm.at[cbase + j], wr.at[s], isem)

        def idx_drain(j):
            s = lax.rem(j, IR)
            pltpu.make_async_copy(src_hbm.at[cbase + j], srcr.at[s], isem).wait()
            pltpu.make_async_copy(dst_hbm.at[cbase + j], dstr.at[s], isem).wait()
            pltpu.make_async_copy(w_hbm.at[cbase + j], wr.at[s], isem).wait()

        def start_gather(j, b):
            s = lax.rem(j, IR)
            pltpu.async_copy(h2_hbm.at[srcr.at[s]], rows.at[b], gsems[b])

        def wait_gather(j, b):
            s = lax.rem(j, IR)
            pltpu.make_async_copy(h2_hbm.at[srcr.at[s]], rows.at[b], gsems[b]).wait()

        def start_scatter(j, b):
            s = lax.rem(j, IR)
            pltpu.async_copy(rows.at[b], acc.at[dstr.at[s]], ssems[b], add=True)

        def wait_scatter(j, b):
            s = lax.rem(j, IR)
            pltpu.make_async_copy(rows.at[b], acc.at[dstr.at[s]], ssems[b]).wait()

        def scale(j, b):
            s = lax.rem(j, IR)

            def row(r, _):
                wspl = plsc.load_gather(wr, [zidx + s, zidx + r])
                for cc in range(D // LANES):
                    sl = pl.ds(cc * LANES, LANES)
                    rows[b, r, sl] = rows[b, r, sl] * wspl
                return 0

            lax.fori_loop(0, C, row, 0)

        def zr(r, _):
            for cc in range(D // LANES):
                rows[0, r, pl.ds(cc * LANES, LANES)] = zero16
            return 0

        lax.fori_loop(0, C, zr, 0)
        for k in range(RPS // C):
            pltpu.sync_copy(rows.at[0], acc.at[pl.ds(rbase + k * C, C)])
        plsc.subcore_barrier()

        for j in range(6):
            idx_prefetch(j)
        idx_drain(0)
        start_gather(0, 0)
        idx_drain(1)
        start_gather(1, 1)

        def outer(jj, _):
            j0 = jj * NB
            for u in range(NB):
                j = j0 + u
                b = u
                wait_gather(j, b)
                scale(j, b)
                start_scatter(j, b)
                jg = j + 2
                bg = (u + 2) % NB

                @pl.when(jg < nch)
                def _():
                    @pl.when(jg >= NB)
                    def _():
                        wait_scatter(jg - NB, bg)

                    idx_drain(jg)
                    start_gather(jg, bg)

                jp = j + 6

                @pl.when(jp < nch)
                def _():
                    idx_prefetch(jp)
            return 0

        lax.fori_loop(0, nch // NB, outer, 0)
        for u in range(NB):
            wait_scatter(nch - NB + u, u)
        plsc.subcore_barrier()

        def cp(k, _):
            pltpu.sync_copy(acc.at[pl.ds(rbase + k * C, C)], rows.at[0])
            pltpu.sync_copy(rows.at[0], out_hbm.at[cid, pl.ds(rbase + k * C, C)])
            return 0

        lax.fori_loop(0, RPS // C, cp, 0)

    pl.run_scoped(
        inner,
        pltpu.VMEM((NB, C, D), jnp.float32),
        pltpu.VMEM((IR, C), jnp.int32),
        pltpu.VMEM((IR, C), jnp.int32),
        pltpu.VMEM((IR, C), jnp.float32),
    )


_agg_kernel = pl.kernel(
    _agg_body,
    out_type=jax.ShapeDtypeStruct((NC, NP, D), jnp.float32),
    mesh=plsc.VectorSubcoreMesh(core_axis_name="c", subcore_axis_name="s"),
    scratch_types=[
        pltpu.VMEM_SHARED((NP, D), jnp.float32),
        pltpu.SemaphoreType.DMA,
        pltpu.SemaphoreType.DMA,
        pltpu.SemaphoreType.DMA,
        pltpu.SemaphoreType.DMA,
        pltpu.SemaphoreType.DMA,
        pltpu.SemaphoreType.DMA,
        pltpu.SemaphoreType.DMA,
        pltpu.SemaphoreType.DMA,
        pltpu.SemaphoreType.DMA,
    ],
    compiler_params=_SC_PARAMS,
)

BR = 2048
_GRID = NP // BR


def _h2_body(deg32_ref, x_ref, w_ref, h2_ref):
    deg = jnp.sum(deg32_ref[...], axis=0) + 1.0
    dis = lax.rsqrt(deg)
    h = jnp.dot(x_ref[...], w_ref[...], preferred_element_type=jnp.float32)
    h2_ref[...] = h * dis[:, None]


_h2_call = pl.pallas_call(
    _h2_body,
    grid=(_GRID,),
    in_specs=[
        pl.BlockSpec((NW, BR), lambda i: (0, i)),
        pl.BlockSpec((BR, D), lambda i: (i, 0)),
        pl.BlockSpec((D, D), lambda i: (0, 0)),
    ],
    out_specs=pl.BlockSpec((BR, D), lambda i: (i, 0)),
    out_shape=jax.ShapeDtypeStruct((NP, D), jnp.float32),
)


def _out_body(p_ref, h2_ref, deg32_ref, b_ref, o_ref):
    deg = jnp.sum(deg32_ref[...], axis=0) + 1.0
    dis = lax.rsqrt(deg)
    agg = p_ref[0] + p_ref[1] + h2_ref[...]
    o_ref[...] = jnp.maximum(agg * dis[:, None] + b_ref[...], 0.0)


_out_call = pl.pallas_call(
    _out_body,
    grid=(_GRID,),
    in_specs=[
        pl.BlockSpec((NC, BR, D), lambda i: (0, i, 0)),
        pl.BlockSpec((BR, D), lambda i: (i, 0)),
        pl.BlockSpec((NW, BR), lambda i: (0, i)),
        pl.BlockSpec((1, D), lambda i: (0, 0)),
    ],
    out_specs=pl.BlockSpec((BR, D), lambda i: (i, 0)),
    out_shape=jax.ShapeDtypeStruct((NP, D), jnp.float32),
)


def kernel(x, edge_index, edge_weight, W, b, item_embeddings):
    pad = EPAD - E
    src_p = jnp.pad(edge_index[0], (0, pad)).reshape(TCH, C)
    dst_p = jnp.pad(edge_index[1], (0, pad)).reshape(TCH, C)
    w_p = jnp.pad(edge_weight, (0, pad)).reshape(TCH, C)
    x_p = jnp.pad(x, ((0, NP - N_USERS), (0, 0)))
    dst_flat = dst_p.reshape(NW, EPW1)
    w_flat = w_p.reshape(NW, EPW1)
    deg32 = _deg_kernel(dst_flat, w_flat)
    h2 = _h2_call(deg32, x_p, W)
    p = _agg_kernel(h2, src_p, dst_p, w_p)
    user_h = _out_call(p, h2, deg32, b.reshape(1, D))
    return user_h[:N_USERS], item_embeddings
